# baseline (speedup 1.0000x reference)
import jax
import jax.numpy as jnp
from jax.experimental import pallas as pl
from jax.experimental.pallas import tpu as pltpu

_B = 1024
_H = 50
_K = 64
_C = 256
_J = 8
_CJ = _C // _J
_R = _B // _C
_Q = 4096 // _B


def _probe_body(*refs):
    x_refs = refs[:_J]
    seen_ref = refs[_J]
    out_ref = refs[_J + 1]
    acc_ref = refs[_J + 2]
    q = pl.program_id(1)

    ones = jnp.ones((_K, 128), jnp.float32)
    for j in range(_J):
        for t in range(7):
            rows = 8 if t < 6 else 2
            a = jnp.abs(x_refs[j][:, 0, 8 * t : 8 * t + rows, :])
            a = a.reshape(_CJ * rows, _K)
            s = jax.lax.dot_general(
                a, ones, (((1,), (0,)), ((), ())),
                preferred_element_type=jnp.float32,
            )
            rz = (s == 0.0).astype(jnp.float32)

            @pl.when(q == 0)
            def _init():
                acc_ref[t, pl.ds(j * _CJ * rows, _CJ * rows)] = rz

            @pl.when(q > 0)
            def _acc():
                acc_ref[t, pl.ds(j * _CJ * rows, _CJ * rows)] += rz

    @pl.when(q == _Q - 1)
    def _emit():
        pieces = [acc_ref[t].reshape(_C, 8, 128)[:, :, 0] for t in range(6)]
        pieces.append(acc_ref[6, : _C * 2].reshape(_C, 2, 128)[:, :, 0])
        out_ref[...] = seen_ref[...] + jnp.concatenate(pieces, axis=1)


def kernel(x, seen):
    in_specs = [
        pl.BlockSpec(
            (_CJ, 2, _H, _K),
            lambda r, q, j=j: (_J * (r + _R * q) + j, 0, 0, 0),
        )
        for j in range(_J)
    ]
    in_specs.append(pl.BlockSpec((_C, _H), lambda r, q: (r, 0)))
    seen_new = pl.pallas_call(
        _probe_body,
        grid=(_R, _Q),
        in_specs=in_specs,
        out_specs=pl.BlockSpec((_C, _H), lambda r, q: (r, 0)),
        out_shape=jax.ShapeDtypeStruct((_B, _H), jnp.float32),
        scratch_shapes=[pltpu.VMEM((7, _C * 8, 128), jnp.float32)],
    )(*([x] * _J), seen)
    return (x, seen_new)


# speedup vs baseline: 1.8852x; 1.8852x over previous
import jax
import jax.numpy as jnp
from jax.experimental import pallas as pl
from jax.experimental.pallas import tpu as pltpu

_B = 1024
_H = 50
_K = 64
_C = 256
_R = _B // _C
_Q = 4096 // _B


def _probe_body(x_ref, seen_ref, out_ref, acc_ref):
    q = pl.program_id(1)

    ones = jnp.ones((_K, 128), jnp.float32)
    for t in range(7):
        rows = 8 if t < 6 else 2
        a = jnp.abs(x_ref[:, 8 * t : 8 * t + rows, :])
        a = a.reshape(_C * rows, _K)
        s = jax.lax.dot_general(
            a, ones, (((1,), (0,)), ((), ())),
            preferred_element_type=jnp.float32,
        )
        rz = (s == 0.0).astype(jnp.float32)

        @pl.when(q == 0)
        def _init():
            acc_ref[t, : _C * rows] = rz

        @pl.when(q > 0)
        def _acc():
            acc_ref[t, : _C * rows] += rz

    @pl.when(q == _Q - 1)
    def _emit():
        pieces = [acc_ref[t].reshape(_C, 8, 128)[:, :, 0] for t in range(6)]
        pieces.append(acc_ref[6, : _C * 2].reshape(_C, 2, 128)[:, :, 0])
        out_ref[...] = seen_ref[...] + jnp.concatenate(pieces, axis=1)


def kernel(x, seen):
    x0 = x[:, 0]
    seen_new = pl.pallas_call(
        _probe_body,
        grid=(_R, _Q),
        in_specs=[
            pl.BlockSpec((_C, _H, _K), lambda r, q: (r + _R * q, 0, 0)),
            pl.BlockSpec((_C, _H), lambda r, q: (r, 0)),
        ],
        out_specs=pl.BlockSpec((_C, _H), lambda r, q: (r, 0)),
        out_shape=jax.ShapeDtypeStruct((_B, _H), jnp.float32),
        scratch_shapes=[pltpu.VMEM((7, _C * 8, 128), jnp.float32)],
    )(x0, seen)
    return (x, seen_new)
